# baseline (speedup 1.0000x reference)
import functools

import jax
import jax.numpy as jnp
from jax import lax
from jax.experimental import pallas as pl
from jax.experimental.pallas import tpu as pltpu
from jax.experimental.pallas import tpu_sc as plsc


def _sc_gather(hist_idx, user_idx, item_idx, user_table, item_table,
               n_hist, n_side, d):
    info = plsc.get_sparse_core_info()
    nc, ns = info.num_cores, info.num_subcores
    nw = nc * ns
    lanes = 128
    rows_w = n_hist // nw
    streams_w = rows_w // lanes
    s_per_chunk = 10
    chunks = streams_w // s_per_chunk
    chunk_rows = s_per_chunk * lanes
    side_w = n_side // nw // lanes

    mesh = plsc.VectorSubcoreMesh(core_axis_name="c", subcore_axis_name="s")
    f32 = jnp.float32

    @functools.partial(
        pl.kernel,
        out_type=(
            jax.ShapeDtypeStruct((n_hist, d), f32),
            jax.ShapeDtypeStruct((n_side, d), f32),
            jax.ShapeDtypeStruct((n_side, d), f32),
        ),
        mesh=mesh,
        compiler_params=pltpu.CompilerParams(use_tc_tiling_on_sc=False),
        scratch_types=[
            pltpu.VMEM((streams_w, lanes), jnp.int32),
            pltpu.VMEM((chunk_rows, d), f32),
            pltpu.VMEM((1, lanes), jnp.int32),
            pltpu.VMEM((lanes, d), f32),
            pltpu.SemaphoreType.DMA,
        ],
    )
    def gather_kernel(hist_idx_h, user_idx_h, item_idx_h, utab_h, itab_h,
                      hist_out, user_out, item_out,
                      idx_v, rows_v, sidx_v, srows_v, sem):
        wid = lax.axis_index("s") * nc + lax.axis_index("c")

        for tab, idx_h, out in ((utab_h, user_idx_h, user_out),
                                (itab_h, item_idx_h, item_out)):
            pltpu.sync_copy(idx_h.at[wid], sidx_v)
            for j in range(side_w):
                pltpu.async_copy(tab.at[sidx_v.at[j]], srows_v, sem).wait()
                pltpu.sync_copy(
                    srows_v,
                    out.at[pl.ds((wid * side_w + j) * lanes, lanes)])

        pltpu.sync_copy(hist_idx_h.at[wid], idx_v)

        @pl.loop(0, chunks)
        def _chunk(c):
            descs = [
                pltpu.async_copy(
                    itab_h.at[idx_v.at[c * s_per_chunk + jj]],
                    rows_v.at[pl.ds(jj * lanes, lanes)],
                    sem,
                )
                for jj in range(s_per_chunk)
            ]
            for desc in descs:
                desc.wait()
            pltpu.sync_copy(
                rows_v,
                hist_out.at[pl.ds(wid * rows_w + c * chunk_rows, chunk_rows)],
            )

    return gather_kernel(
        hist_idx.reshape(nw, streams_w, lanes),
        user_idx.reshape(nw, side_w, lanes),
        item_idx.reshape(nw, side_w, lanes),
        user_table, item_table)


def _tc_dense(hist3, user_emb, item_emb, w_int, w1u_t, w1h_t, b1r, w2r, b2r):
    n_l, n_b, d = hist3.shape
    f32 = jnp.float32

    def body(hist_ref, u_ref, it_ref, wint_ref, w1u_ref, w1h_ref, b1_ref,
             w2_ref, b2_ref, uout_ref, inter_ref, upart_s):
        step = pl.program_id(0)

        @pl.when(step == 0)
        def _init():
            u = u_ref[...]
            upart_s[...] = (
                jnp.dot(u, w1u_ref[...], preferred_element_type=f32)
                + b1_ref[...]
            )
            t = jnp.dot(u, wint_ref[...], preferred_element_type=f32)
            inter_ref[...] = jnp.sum(t * it_ref[...], axis=1, keepdims=True)
            uout_ref[...] = u

        hist = hist_ref[0]
        h = jnp.maximum(
            jnp.dot(hist, w1h_ref[...], preferred_element_type=f32)
            + upart_s[...],
            0.0,
        )
        a = jax.nn.sigmoid(
            jnp.sum(h * w2_ref[...], axis=1, keepdims=True) + b2_ref[0, 0]
        )
        uout_ref[...] += a * hist

    full = lambda shape: pl.BlockSpec(shape, lambda l: (0,) * len(shape))
    return pl.pallas_call(
        body,
        grid=(n_l,),
        in_specs=[
            pl.BlockSpec((1, n_b, d), lambda l: (l, 0, 0)),
            full((n_b, d)),
            full((n_b, d)),
            full((d, d)),
            full((d, d)),
            full((d, d)),
            full((1, d)),
            full((1, d)),
            full((1, 1)),
        ],
        out_specs=[full((n_b, d)), full((n_b, 1))],
        out_shape=[
            jax.ShapeDtypeStruct((n_b, d), f32),
            jax.ShapeDtypeStruct((n_b, 1), f32),
        ],
        scratch_shapes=[pltpu.VMEM((n_b, d), f32)],
    )(hist3, user_emb, item_emb, w_int, w1u_t, w1h_t, b1r, w2r, b2r)


def kernel(user_ids, item_ids, user_history, user_table, item_table,
           W_int, w1, b1, w2, b2):
    n_b, n_l = user_history.shape
    d = user_table.shape[1]
    h_dim = w1.shape[0]

    hist_idx = user_history.T.reshape(-1)
    hist_rows, user_emb, item_emb = _sc_gather(
        hist_idx, user_ids, item_ids,
        user_table, item_table, n_l * n_b, n_b, d)

    user_out, interaction = _tc_dense(
        hist_rows.reshape(n_l, n_b, d), user_emb, item_emb,
        W_int, w1[:, :d].T, w1[:, d:].T,
        b1.reshape(1, h_dim), w2.reshape(1, h_dim),
        b2.reshape(1, 1).astype(jnp.float32))

    return (user_out, item_emb, interaction)
